# Initial kernel scaffold; baseline (speedup 1.0000x reference)
import jax
import jax.numpy as jnp
from jax.experimental import pallas as pl


def kernel(x, edge_index, W_l, b_l, W_r, ln_gamma, ln_beta):
    raise NotImplementedError("write your pallas kernel here")


# speedup vs baseline: 1.1928x; 1.1928x over previous
import functools

import jax
import jax.numpy as jnp
from jax import lax
from jax.experimental import pallas as pl
from jax.experimental.pallas import tpu as pltpu
from jax.experimental.pallas import tpu_sc as plsc

_K = 80
_IB = 16


@functools.lru_cache(maxsize=None)
def _make_sc_agg(N, E, D):
    try:
        info = plsc.get_sparse_core_info()
        NC, NS = info.num_cores, info.num_subcores
    except ValueError:
        NC, NS = 2, 16
    H = D // 2

    e_pt = E // NS
    NCH = -(-e_pt // _K)
    NCH += (-NCH) % _IB
    e_dt = E // (2 * NS)
    NCH2 = -(-e_dt // _K)
    NCH2 += (-NCH2) % _IB
    ZR = -(-N // NS)
    ZR += (-ZR) % _K
    NPAD = ZR * NS
    NP8 = NPAD // 8
    ZR8 = NP8 // NS

    mesh = plsc.VectorSubcoreMesh(core_axis_name="c", subcore_axis_name="s",
                                  num_cores=NC, num_subcores=NS)

    @functools.partial(
        pl.kernel,
        out_type=[
            jax.ShapeDtypeStruct((2, NPAD, H), jnp.float32),
            jax.ShapeDtypeStruct((2, NP8, 128), jnp.float32),
        ],
        mesh=mesh,
        scratch_types=[
            pltpu.VMEM((_IB, _K), jnp.int32),
            pltpu.VMEM((_IB, _K), jnp.int32),
            pltpu.VMEM((_K, H), jnp.float32),
            pltpu.VMEM_SHARED((NPAD, H), jnp.float32),
            pltpu.VMEM_SHARED((NP8, 128), jnp.float32),
            pltpu.SemaphoreType.DMA,
        ],
    )
    def sc_agg(x2, pat, srcs, dsts, patd, rowd, zrow, agg_out, deg_out,
               ia_v, ib_v, buf, agg_sh, deg_sh, sem):
        c = lax.axis_index("c")
        s = lax.axis_index("s")
        NZ = ZR // _K

        pltpu.sync_copy(zrow, buf)

        def zero_blk(i, carry):
            pltpu.sync_copy(buf, agg_sh.at[pl.ds(s * ZR + i * _K, _K)])
            return carry

        lax.fori_loop(0, NZ, zero_blk, 0)
        pltpu.sync_copy(buf, deg_sh.at[pl.ds(s * ZR8, ZR8)])

        plsc.subcore_barrier()

        def fgroup(g, carry):
            pltpu.sync_copy(srcs.at[c, s, pl.ds(g * _IB, _IB)], ia_v)
            pltpu.sync_copy(dsts.at[s, pl.ds(g * _IB, _IB)], ib_v)

            def chunk(j, carry2):
                pltpu.async_copy(x2.at[ia_v.at[j]], buf, sem).wait()
                pltpu.sync_copy(buf, agg_sh.at[ib_v.at[j]], add=True)
                return carry2

            lax.fori_loop(0, _IB, chunk, 0)
            return carry

        lax.fori_loop(0, NCH // _IB, fgroup, 0)

        def dgroup(g, carry):
            pltpu.sync_copy(patd.at[c, s, pl.ds(g * _IB, _IB)], ia_v)
            pltpu.sync_copy(rowd.at[c, s, pl.ds(g * _IB, _IB)], ib_v)

            def chunk(j, carry2):
                pltpu.async_copy(pat.at[ia_v.at[j]], buf, sem).wait()
                pltpu.sync_copy(buf, deg_sh.at[ib_v.at[j]], add=True)
                return carry2

            lax.fori_loop(0, _IB, chunk, 0)
            return carry

        lax.fori_loop(0, NCH2 // _IB, dgroup, 0)

        plsc.subcore_barrier()

        def out_blk(i, carry):
            off = s * ZR + i * _K
            pltpu.sync_copy(agg_sh.at[pl.ds(off, _K)], buf)
            pltpu.sync_copy(buf, agg_out.at[c, pl.ds(off, _K)])
            return carry

        lax.fori_loop(0, NZ, out_blk, 0)
        pltpu.sync_copy(deg_sh.at[pl.ds(s * ZR8, ZR8)], buf)
        pltpu.sync_copy(buf, deg_out.at[c, pl.ds(s * ZR8, ZR8)])

    return sc_agg, NCH, NCH2, ZR, NPAD


def _tc_tail_body(alo_ref, ahi_ref, d0_ref, d1_ref, x_ref, wl_ref, bl_ref,
                  wr_ref, g_ref, b_ref, o_ref):
    deg = d0_ref[0, :, 0:1] + d1_ref[0, :, 0:1]
    inv = 1.0 / jnp.maximum(deg, 1.0)
    h = x_ref.shape[1] // 2
    alo = alo_ref[0] * inv
    ahi = ahi_ref[0] * inv
    out = (jnp.dot(alo, wl_ref[0:h, :], preferred_element_type=jnp.float32)
           + jnp.dot(ahi, wl_ref[h:, :], preferred_element_type=jnp.float32)
           + jnp.dot(x_ref[...], wr_ref[...], preferred_element_type=jnp.float32)
           + bl_ref[...])
    nrm = jnp.sqrt(jnp.sum(out * out, axis=1, keepdims=True))
    out = jnp.maximum(out / jnp.maximum(nrm, 1e-12), 0.0)
    hres = out + x_ref[...]
    mu = jnp.mean(hres, axis=1, keepdims=True)
    hc = hres - mu
    var = jnp.mean(hc * hc, axis=1, keepdims=True)
    o_ref[...] = hc * lax.rsqrt(var + 1e-5) * g_ref[...] + b_ref[...]


def _tc_tail(agg2, deg2, x, W_l, b_l, W_r, gamma, beta):
    N, D = x.shape
    H = D // 2
    BN = 1000
    grid = (N // BN,)
    return pl.pallas_call(
        _tc_tail_body,
        grid=grid,
        in_specs=[
            pl.BlockSpec((1, BN, H), lambda i: (0, i, 0)),
            pl.BlockSpec((1, BN, H), lambda i: (1, i, 0)),
            pl.BlockSpec((1, BN, 16), lambda i: (0, i, 0)),
            pl.BlockSpec((1, BN, 16), lambda i: (1, i, 0)),
            pl.BlockSpec((BN, D), lambda i: (i, 0)),
            pl.BlockSpec((D, D), lambda i: (0, 0)),
            pl.BlockSpec((1, D), lambda i: (0, 0)),
            pl.BlockSpec((D, D), lambda i: (0, 0)),
            pl.BlockSpec((1, D), lambda i: (0, 0)),
            pl.BlockSpec((1, D), lambda i: (0, 0)),
        ],
        out_specs=pl.BlockSpec((BN, D), lambda i: (i, 0)),
        out_shape=jax.ShapeDtypeStruct((N, D), jnp.float32),
    )(agg2, agg2, deg2, deg2, x, W_l, b_l.reshape(1, D), W_r,
      gamma.reshape(1, D), beta.reshape(1, D))


def kernel(x, edge_index, W_l, b_l, W_r, ln_gamma, ln_beta):
    N, D = x.shape
    E = edge_index.shape[1]
    H = D // 2
    NS = 16

    sc_agg, NCH, NCH2, ZR, NPAD = _make_sc_agg(N, E, D)
    e_pt = E // NS
    e_dt = E // (2 * NS)
    EP = NCH * _K
    EP2 = NCH2 * _K

    x2 = jnp.concatenate([x[:, :H], x[:, H:]], axis=0)
    pat = jnp.repeat(jnp.eye(8, dtype=jnp.float32), 16, axis=1)

    src = edge_index[0].reshape(NS, e_pt)
    dst = edge_index[1].reshape(NS, e_pt)
    src_p = jnp.pad(src, ((0, 0), (0, EP - e_pt)))
    dst_p = jnp.pad(dst, ((0, 0), (0, EP - e_pt)), constant_values=N)
    srcs = jnp.stack([src_p, src_p + N]).reshape(2, NS, NCH, _K)
    dsts = dst_p.reshape(NS, NCH, _K)

    dd = edge_index[1].reshape(2, NS, e_dt)
    dd_p = jnp.pad(dd, ((0, 0), (0, 0), (0, EP2 - e_dt)), constant_values=N)
    patd = (dd_p % 8).reshape(2, NS, NCH2, _K)
    rowd = (dd_p // 8).reshape(2, NS, NCH2, _K)

    zrow = jnp.zeros((_K, H), jnp.float32)

    agg2, deg8 = sc_agg(x2, pat, srcs, dsts, patd, rowd, zrow)
    deg2 = deg8.reshape(2, NPAD, 16)
    return _tc_tail(agg2, deg2, x, W_l, b_l, W_r, ln_gamma, ln_beta)


# speedup vs baseline: 1.2412x; 1.0406x over previous
import functools

import jax
import jax.numpy as jnp
from jax import lax
from jax.experimental import pallas as pl
from jax.experimental.pallas import tpu as pltpu
from jax.experimental.pallas import tpu_sc as plsc

_K = 80
_IB = 16


@functools.lru_cache(maxsize=None)
def _make_sc_agg(N, E, D):
    try:
        info = plsc.get_sparse_core_info()
        NC, NS = info.num_cores, info.num_subcores
    except ValueError:
        NC, NS = 2, 16
    H = D // 2

    e_pt = E // NS
    NCH = -(-e_pt // _K)
    NCH += (-NCH) % _IB
    e_dt = E // (2 * NS)
    NCH2 = -(-e_dt // _K)
    NCH2 += (-NCH2) % _IB
    ZR = -(-N // NS)
    ZR += (-ZR) % _K
    NPAD = ZR * NS
    NP8 = NPAD // 8
    ZR8 = NP8 // NS

    mesh = plsc.VectorSubcoreMesh(core_axis_name="c", subcore_axis_name="s",
                                  num_cores=NC, num_subcores=NS)

    @functools.partial(
        pl.kernel,
        out_type=[
            jax.ShapeDtypeStruct((2, NPAD, H), jnp.float32),
            jax.ShapeDtypeStruct((2, NP8, 128), jnp.float32),
        ],
        mesh=mesh,
        scratch_types=[
            pltpu.VMEM((_IB, _K), jnp.int32),
            pltpu.VMEM((_IB, _K), jnp.int32),
            pltpu.VMEM((_K, H), jnp.float32),
            pltpu.VMEM((_K, H), jnp.float32),
            pltpu.VMEM_SHARED((NPAD, H), jnp.float32),
            pltpu.VMEM_SHARED((NP8, 128), jnp.float32),
            pltpu.SemaphoreType.DMA,
            pltpu.SemaphoreType.DMA,
            pltpu.SemaphoreType.DMA,
            pltpu.SemaphoreType.DMA,
        ],
    )
    def sc_agg(x2, pat, srcs, dsts, patd, rowd, zrow, agg_out, deg_out,
               ia_v, ib_v, bufa, bufb, agg_sh, deg_sh, ga, gb, sa, sb):
        buf = bufa
        c = lax.axis_index("c")
        s = lax.axis_index("s")
        NZ = ZR // _K

        pltpu.sync_copy(zrow, buf)

        def zero_blk(i, carry):
            pltpu.sync_copy(buf, agg_sh.at[pl.ds(s * ZR + i * _K, _K)])
            return carry

        lax.fori_loop(0, NZ, zero_blk, 0)
        pltpu.sync_copy(buf, deg_sh.at[pl.ds(s * ZR8, ZR8)])

        plsc.subcore_barrier()

        bufs = (bufa, bufb)
        gsem = (ga, gb)
        ssem = (sa, sb)

        def run_group(tab, acc):
            def gath(p, b):
                return pltpu.make_async_copy(tab.at[ia_v.at[p]], bufs[b],
                                             gsem[b])

            def scat(p, b):
                return pltpu.make_async_copy(bufs[b], acc.at[ib_v.at[p]],
                                             ssem[b])

            gath(0, 0).start()
            for p in range(_IB):
                b = p & 1
                nb = 1 - b
                gath(p, b).wait()
                scat(p, b).start(add=True)
                if p + 1 < _IB:
                    if p >= 1:
                        scat(p - 1, nb).wait()
                    gath(p + 1, nb).start()
            scat(_IB - 2, (_IB - 2) & 1).wait()
            scat(_IB - 1, (_IB - 1) & 1).wait()

        def fgroup(g, carry):
            pltpu.sync_copy(srcs.at[c, s, pl.ds(g * _IB, _IB)], ia_v)
            pltpu.sync_copy(dsts.at[s, pl.ds(g * _IB, _IB)], ib_v)
            run_group(x2, agg_sh)
            return carry

        lax.fori_loop(0, NCH // _IB, fgroup, 0)

        def dgroup(g, carry):
            pltpu.sync_copy(patd.at[c, s, pl.ds(g * _IB, _IB)], ia_v)
            pltpu.sync_copy(rowd.at[c, s, pl.ds(g * _IB, _IB)], ib_v)
            run_group(pat, deg_sh)
            return carry

        lax.fori_loop(0, NCH2 // _IB, dgroup, 0)

        plsc.subcore_barrier()

        def out_blk(i, carry):
            off = s * ZR + i * _K
            pltpu.sync_copy(agg_sh.at[pl.ds(off, _K)], buf)
            pltpu.sync_copy(buf, agg_out.at[c, pl.ds(off, _K)])
            return carry

        lax.fori_loop(0, NZ, out_blk, 0)
        pltpu.sync_copy(deg_sh.at[pl.ds(s * ZR8, ZR8)], buf)
        pltpu.sync_copy(buf, deg_out.at[c, pl.ds(s * ZR8, ZR8)])

    return sc_agg, NCH, NCH2, ZR, NPAD


def _tc_tail_body(alo_ref, ahi_ref, d0_ref, d1_ref, x_ref, wl_ref, bl_ref,
                  wr_ref, g_ref, b_ref, o_ref):
    deg = d0_ref[0, :, 0:1] + d1_ref[0, :, 0:1]
    inv = 1.0 / jnp.maximum(deg, 1.0)
    h = x_ref.shape[1] // 2
    alo = alo_ref[0] * inv
    ahi = ahi_ref[0] * inv
    out = (jnp.dot(alo, wl_ref[0:h, :], preferred_element_type=jnp.float32)
           + jnp.dot(ahi, wl_ref[h:, :], preferred_element_type=jnp.float32)
           + jnp.dot(x_ref[...], wr_ref[...], preferred_element_type=jnp.float32)
           + bl_ref[...])
    nrm = jnp.sqrt(jnp.sum(out * out, axis=1, keepdims=True))
    out = jnp.maximum(out / jnp.maximum(nrm, 1e-12), 0.0)
    hres = out + x_ref[...]
    mu = jnp.mean(hres, axis=1, keepdims=True)
    hc = hres - mu
    var = jnp.mean(hc * hc, axis=1, keepdims=True)
    o_ref[...] = hc * lax.rsqrt(var + 1e-5) * g_ref[...] + b_ref[...]


def _tc_tail(agg2, deg2, x, W_l, b_l, W_r, gamma, beta):
    N, D = x.shape
    H = D // 2
    BN = 1000
    grid = (N // BN,)
    return pl.pallas_call(
        _tc_tail_body,
        grid=grid,
        in_specs=[
            pl.BlockSpec((1, BN, H), lambda i: (0, i, 0)),
            pl.BlockSpec((1, BN, H), lambda i: (1, i, 0)),
            pl.BlockSpec((1, BN, 16), lambda i: (0, i, 0)),
            pl.BlockSpec((1, BN, 16), lambda i: (1, i, 0)),
            pl.BlockSpec((BN, D), lambda i: (i, 0)),
            pl.BlockSpec((D, D), lambda i: (0, 0)),
            pl.BlockSpec((1, D), lambda i: (0, 0)),
            pl.BlockSpec((D, D), lambda i: (0, 0)),
            pl.BlockSpec((1, D), lambda i: (0, 0)),
            pl.BlockSpec((1, D), lambda i: (0, 0)),
        ],
        out_specs=pl.BlockSpec((BN, D), lambda i: (i, 0)),
        out_shape=jax.ShapeDtypeStruct((N, D), jnp.float32),
    )(agg2, agg2, deg2, deg2, x, W_l, b_l.reshape(1, D), W_r,
      gamma.reshape(1, D), beta.reshape(1, D))


def kernel(x, edge_index, W_l, b_l, W_r, ln_gamma, ln_beta):
    N, D = x.shape
    E = edge_index.shape[1]
    H = D // 2
    NS = 16

    sc_agg, NCH, NCH2, ZR, NPAD = _make_sc_agg(N, E, D)
    e_pt = E // NS
    e_dt = E // (2 * NS)
    EP = NCH * _K
    EP2 = NCH2 * _K

    x2 = jnp.concatenate([x[:, :H], x[:, H:]], axis=0)
    pat = jnp.repeat(jnp.eye(8, dtype=jnp.float32), 16, axis=1)

    src = edge_index[0].reshape(NS, e_pt)
    dst = edge_index[1].reshape(NS, e_pt)
    src_p = jnp.pad(src, ((0, 0), (0, EP - e_pt)))
    dst_p = jnp.pad(dst, ((0, 0), (0, EP - e_pt)), constant_values=N)
    srcs = jnp.stack([src_p, src_p + N]).reshape(2, NS, NCH, _K)
    dsts = dst_p.reshape(NS, NCH, _K)

    dd = edge_index[1].reshape(2, NS, e_dt)
    dd_p = jnp.pad(dd, ((0, 0), (0, 0), (0, EP2 - e_dt)), constant_values=N)
    patd = (dd_p % 8).reshape(2, NS, NCH2, _K)
    rowd = (dd_p // 8).reshape(2, NS, NCH2, _K)

    zrow = jnp.zeros((_K, H), jnp.float32)

    agg2, deg8 = sc_agg(x2, pat, srcs, dsts, patd, rowd, zrow)
    deg2 = deg8.reshape(2, NPAD, 16)
    return _tc_tail(agg2, deg2, x, W_l, b_l, W_r, ln_gamma, ln_beta)


# speedup vs baseline: 1.8044x; 1.4537x over previous
import functools

import jax
import jax.numpy as jnp
from jax import lax
from jax.experimental import pallas as pl
from jax.experimental.pallas import tpu as pltpu
from jax.experimental.pallas import tpu_sc as plsc

_K = 128
_IB = 8
_PN = 16


@functools.lru_cache(maxsize=None)
def _make_sc_agg(N, E, D):
    try:
        info = plsc.get_sparse_core_info()
        NC, NS = info.num_cores, info.num_subcores
    except ValueError:
        NC, NS = 2, 16
    H = D // 2

    e_pt = E // NS
    NCH = -(-e_pt // _K)
    NCH += (-NCH) % _IB
    e_dt = E // (2 * NS)
    NCH2 = -(-e_dt // _K)
    NCH2 += (-NCH2) % _IB
    ZR = -(-N // NS)
    ZR += (-ZR) % _K
    NPAD = ZR * NS
    NPP = NPAD // _PN
    ZRP = NPP // NS

    mesh = plsc.VectorSubcoreMesh(core_axis_name="c", subcore_axis_name="s",
                                  num_cores=NC, num_subcores=NS)

    @functools.partial(
        pl.kernel,
        out_type=[
            jax.ShapeDtypeStruct((2, NPAD, H), jnp.float32),
            jax.ShapeDtypeStruct((2, NPP, 128), jnp.float32),
        ],
        mesh=mesh,
        scratch_types=[
            pltpu.VMEM((_IB, _K), jnp.int32),
            pltpu.VMEM((_IB, _K), jnp.int32),
            pltpu.VMEM((_K, H), jnp.float32),
            pltpu.VMEM((_K, H), jnp.float32),
            pltpu.VMEM_SHARED((NPAD, H), jnp.float32),
            pltpu.VMEM_SHARED((NPP, 128), jnp.float32),
            pltpu.SemaphoreType.DMA,
            pltpu.SemaphoreType.DMA,
            pltpu.SemaphoreType.DMA,
            pltpu.SemaphoreType.DMA,
        ],
    )
    def sc_agg(x2, pat, srcs, dsts, patd, rowd, zrow, agg_out, deg_out,
               ia_v, ib_v, bufa, bufb, agg_sh, deg_sh, ga, gb, sa, sb):
        buf = bufa
        c = lax.axis_index("c")
        s = lax.axis_index("s")
        NZ = ZR // _K

        pltpu.sync_copy(zrow, buf)

        def zero_blk(i, carry):
            pltpu.sync_copy(buf, agg_sh.at[pl.ds(s * ZR + i * _K, _K)])
            return carry

        lax.fori_loop(0, NZ, zero_blk, 0)
        pltpu.sync_copy(buf.at[pl.ds(0, ZRP)], deg_sh.at[pl.ds(s * ZRP, ZRP)])

        plsc.subcore_barrier()

        bufs = (bufa, bufb)
        gsem = (ga, gb)
        ssem = (sa, sb)

        def run_group(tab, acc):
            def gath(p, b):
                return pltpu.make_async_copy(tab.at[ia_v.at[p]], bufs[b],
                                             gsem[b])

            def scat(p, b):
                return pltpu.make_async_copy(bufs[b], acc.at[ib_v.at[p]],
                                             ssem[b])

            gath(0, 0).start()
            for p in range(_IB):
                b = p & 1
                nb = 1 - b
                gath(p, b).wait()
                scat(p, b).start(add=True)
                if p + 1 < _IB:
                    if p >= 1:
                        scat(p - 1, nb).wait()
                    gath(p + 1, nb).start()
            scat(_IB - 2, (_IB - 2) & 1).wait()
            scat(_IB - 1, (_IB - 1) & 1).wait()

        def fgroup(g, carry):
            pltpu.sync_copy(srcs.at[c, s, pl.ds(g * _IB, _IB)], ia_v)
            pltpu.sync_copy(dsts.at[s, pl.ds(g * _IB, _IB)], ib_v)
            run_group(x2, agg_sh)
            return carry

        lax.fori_loop(0, NCH // _IB, fgroup, 0)

        def dgroup(g, carry):
            pltpu.sync_copy(patd.at[c, s, pl.ds(g * _IB, _IB)], ia_v)
            pltpu.sync_copy(rowd.at[c, s, pl.ds(g * _IB, _IB)], ib_v)
            run_group(pat, deg_sh)
            return carry

        lax.fori_loop(0, NCH2 // _IB, dgroup, 0)

        plsc.subcore_barrier()

        def out_blk(i, carry):
            off = s * ZR + i * _K
            pltpu.sync_copy(agg_sh.at[pl.ds(off, _K)], buf)
            pltpu.sync_copy(buf, agg_out.at[c, pl.ds(off, _K)])
            return carry

        lax.fori_loop(0, NZ, out_blk, 0)
        pltpu.sync_copy(deg_sh.at[pl.ds(s * ZRP, ZRP)], buf.at[pl.ds(0, ZRP)])
        pltpu.sync_copy(buf.at[pl.ds(0, ZRP)], deg_out.at[c, pl.ds(s * ZRP, ZRP)])

    return sc_agg, NCH, NCH2, ZR, NPAD


def _tc_tail_body(alo_ref, ahi_ref, d0_ref, d1_ref, x_ref, wl_ref, bl_ref,
                  wr_ref, g_ref, b_ref, o_ref):
    deg = d0_ref[0, :, 0:1] + d1_ref[0, :, 0:1]
    inv = 1.0 / jnp.maximum(deg, 1.0)
    h = x_ref.shape[1] // 2
    alo = alo_ref[0] * inv
    ahi = ahi_ref[0] * inv
    out = (jnp.dot(alo, wl_ref[0:h, :], preferred_element_type=jnp.float32)
           + jnp.dot(ahi, wl_ref[h:, :], preferred_element_type=jnp.float32)
           + jnp.dot(x_ref[...], wr_ref[...], preferred_element_type=jnp.float32)
           + bl_ref[...])
    nrm = jnp.sqrt(jnp.sum(out * out, axis=1, keepdims=True))
    out = jnp.maximum(out / jnp.maximum(nrm, 1e-12), 0.0)
    hres = out + x_ref[...]
    mu = jnp.mean(hres, axis=1, keepdims=True)
    hc = hres - mu
    var = jnp.mean(hc * hc, axis=1, keepdims=True)
    o_ref[...] = hc * lax.rsqrt(var + 1e-5) * g_ref[...] + b_ref[...]


def _tc_tail(agg2, deg2, x, W_l, b_l, W_r, gamma, beta):
    N, D = x.shape
    H = D // 2
    BN = 1000
    grid = (N // BN,)
    return pl.pallas_call(
        _tc_tail_body,
        grid=grid,
        in_specs=[
            pl.BlockSpec((1, BN, H), lambda i: (0, i, 0)),
            pl.BlockSpec((1, BN, H), lambda i: (1, i, 0)),
            pl.BlockSpec((1, BN, 8), lambda i: (0, i, 0)),
            pl.BlockSpec((1, BN, 8), lambda i: (1, i, 0)),
            pl.BlockSpec((BN, D), lambda i: (i, 0)),
            pl.BlockSpec((D, D), lambda i: (0, 0)),
            pl.BlockSpec((1, D), lambda i: (0, 0)),
            pl.BlockSpec((D, D), lambda i: (0, 0)),
            pl.BlockSpec((1, D), lambda i: (0, 0)),
            pl.BlockSpec((1, D), lambda i: (0, 0)),
        ],
        out_specs=pl.BlockSpec((BN, D), lambda i: (i, 0)),
        out_shape=jax.ShapeDtypeStruct((N, D), jnp.float32),
    )(agg2, agg2, deg2, deg2, x, W_l, b_l.reshape(1, D), W_r,
      gamma.reshape(1, D), beta.reshape(1, D))


def kernel(x, edge_index, W_l, b_l, W_r, ln_gamma, ln_beta):
    N, D = x.shape
    E = edge_index.shape[1]
    H = D // 2
    NS = 16

    sc_agg, NCH, NCH2, ZR, NPAD = _make_sc_agg(N, E, D)
    e_pt = E // NS
    e_dt = E // (2 * NS)
    EP = NCH * _K
    EP2 = NCH2 * _K

    x2 = jnp.concatenate([x[:, :H], x[:, H:]], axis=0)
    pat = jnp.repeat(jnp.eye(_PN, dtype=jnp.float32), 128 // _PN, axis=1)

    src = edge_index[0].reshape(NS, e_pt)
    dst = edge_index[1].reshape(NS, e_pt)
    src_p = jnp.pad(src, ((0, 0), (0, EP - e_pt)))
    dst_p = jnp.pad(dst, ((0, 0), (0, EP - e_pt)), constant_values=N)
    srcs = jnp.stack([src_p, src_p + N]).reshape(2, NS, NCH, _K)
    dsts = dst_p.reshape(NS, NCH, _K)

    dd = edge_index[1].reshape(2, NS, e_dt)
    dd_p = jnp.pad(dd, ((0, 0), (0, 0), (0, EP2 - e_dt)), constant_values=N)
    patd = (dd_p % _PN).reshape(2, NS, NCH2, _K)
    rowd = (dd_p // _PN).reshape(2, NS, NCH2, _K)

    zrow = jnp.zeros((_K, H), jnp.float32)

    agg2, deg8 = sc_agg(x2, pat, srcs, dsts, patd, rowd, zrow)
    deg2 = deg8.reshape(2, NPAD, 128 // _PN)
    return _tc_tail(agg2, deg2, x, W_l, b_l, W_r, ln_gamma, ln_beta)


# speedup vs baseline: 1.8132x; 1.0049x over previous
import functools

import jax
import jax.numpy as jnp
from jax import lax
from jax.experimental import pallas as pl
from jax.experimental.pallas import tpu as pltpu
from jax.experimental.pallas import tpu_sc as plsc

_K = 128
_IB = 8
_IBF = 16
_PN = 16


@functools.lru_cache(maxsize=None)
def _make_sc_agg(N, E, D):
    try:
        info = plsc.get_sparse_core_info()
        NC, NS = info.num_cores, info.num_subcores
    except ValueError:
        NC, NS = 2, 16
    H = D // 2

    e_pt = E // NS
    NCH = -(-e_pt // _K)
    NCH += (-NCH) % _IBF
    e_dt = E // (2 * NS)
    NCH2 = -(-e_dt // _K)
    NCH2 += (-NCH2) % _IB
    ZR = -(-N // NS)
    ZR += (-ZR) % _K
    NPAD = ZR * NS
    NPP = NPAD // _PN
    ZRP = NPP // NS

    mesh = plsc.VectorSubcoreMesh(core_axis_name="c", subcore_axis_name="s",
                                  num_cores=NC, num_subcores=NS)

    @functools.partial(
        pl.kernel,
        out_type=[
            jax.ShapeDtypeStruct((2, NPAD, H), jnp.float32),
            jax.ShapeDtypeStruct((2, NPP, 128), jnp.float32),
        ],
        mesh=mesh,
        scratch_types=[
            pltpu.VMEM((_IBF, _K), jnp.int32),
            pltpu.VMEM((_IBF, _K), jnp.int32),
            pltpu.VMEM((_K, H), jnp.float32),
            pltpu.VMEM((_K, H), jnp.float32),
            pltpu.VMEM_SHARED((NPAD, H), jnp.float32),
            pltpu.VMEM_SHARED((NPP, 128), jnp.float32),
            pltpu.SemaphoreType.DMA,
            pltpu.SemaphoreType.DMA,
            pltpu.SemaphoreType.DMA,
            pltpu.SemaphoreType.DMA,
        ],
    )
    def sc_agg(x2, pat, srcs, dsts, patd, rowd, zrow, agg_out, deg_out,
               ia_v, ib_v, bufa, bufb, agg_sh, deg_sh, ga, gb, sa, sb):
        buf = bufa
        c = lax.axis_index("c")
        s = lax.axis_index("s")
        NZ = ZR // _K

        pltpu.sync_copy(zrow, buf)

        def zero_blk(i, carry):
            pltpu.sync_copy(buf, agg_sh.at[pl.ds(s * ZR + i * _K, _K)])
            return carry

        lax.fori_loop(0, NZ, zero_blk, 0)
        pltpu.sync_copy(buf.at[pl.ds(0, ZRP)], deg_sh.at[pl.ds(s * ZRP, ZRP)])

        plsc.subcore_barrier()

        bufs = (bufa, bufb)
        gsem = (ga, gb)
        ssem = (sa, sb)

        def run_group(tab, acc, n):
            def gath(p, b):
                return pltpu.make_async_copy(tab.at[ia_v.at[p]], bufs[b],
                                             gsem[b])

            def scat(p, b):
                return pltpu.make_async_copy(bufs[b], acc.at[ib_v.at[p]],
                                             ssem[b])

            gath(0, 0).start()
            for p in range(n):
                b = p & 1
                nb = 1 - b
                gath(p, b).wait()
                scat(p, b).start(add=True)
                if p + 1 < n:
                    if p >= 1:
                        scat(p - 1, nb).wait()
                    gath(p + 1, nb).start()
            scat(n - 2, (n - 2) & 1).wait()
            scat(n - 1, (n - 1) & 1).wait()

        def fgroup(g, carry):
            pltpu.sync_copy(srcs.at[c, s, pl.ds(g * _IBF, _IBF)], ia_v)
            pltpu.sync_copy(dsts.at[s, pl.ds(g * _IBF, _IBF)], ib_v)
            run_group(x2, agg_sh, _IBF)
            return carry

        lax.fori_loop(0, NCH // _IBF, fgroup, 0)

        def dgroup(g, carry):
            pltpu.sync_copy(patd.at[c, s, pl.ds(g * _IB, _IB)],
                            ia_v.at[pl.ds(0, _IB)])
            pltpu.sync_copy(rowd.at[c, s, pl.ds(g * _IB, _IB)],
                            ib_v.at[pl.ds(0, _IB)])
            run_group(pat, deg_sh, _IB)
            return carry

        lax.fori_loop(0, NCH2 // _IB, dgroup, 0)

        plsc.subcore_barrier()

        def out_blk(i, carry):
            off = s * ZR + i * _K
            pltpu.sync_copy(agg_sh.at[pl.ds(off, _K)], buf)
            pltpu.sync_copy(buf, agg_out.at[c, pl.ds(off, _K)])
            return carry

        lax.fori_loop(0, NZ, out_blk, 0)
        pltpu.sync_copy(deg_sh.at[pl.ds(s * ZRP, ZRP)], buf.at[pl.ds(0, ZRP)])
        pltpu.sync_copy(buf.at[pl.ds(0, ZRP)], deg_out.at[c, pl.ds(s * ZRP, ZRP)])

    return sc_agg, NCH, NCH2, ZR, NPAD


def _tc_tail_body(alo_ref, ahi_ref, d0_ref, d1_ref, x_ref, wl_ref, bl_ref,
                  wr_ref, g_ref, b_ref, o_ref):
    deg = d0_ref[0, :, 0:1] + d1_ref[0, :, 0:1]
    inv = 1.0 / jnp.maximum(deg, 1.0)
    h = x_ref.shape[1] // 2
    alo = alo_ref[0] * inv
    ahi = ahi_ref[0] * inv
    out = (jnp.dot(alo, wl_ref[0:h, :], preferred_element_type=jnp.float32)
           + jnp.dot(ahi, wl_ref[h:, :], preferred_element_type=jnp.float32)
           + jnp.dot(x_ref[...], wr_ref[...], preferred_element_type=jnp.float32)
           + bl_ref[...])
    nrm = jnp.sqrt(jnp.sum(out * out, axis=1, keepdims=True))
    out = jnp.maximum(out / jnp.maximum(nrm, 1e-12), 0.0)
    hres = out + x_ref[...]
    mu = jnp.mean(hres, axis=1, keepdims=True)
    hc = hres - mu
    var = jnp.mean(hc * hc, axis=1, keepdims=True)
    o_ref[...] = hc * lax.rsqrt(var + 1e-5) * g_ref[...] + b_ref[...]


def _tc_tail(agg2, deg2, x, W_l, b_l, W_r, gamma, beta):
    N, D = x.shape
    H = D // 2
    BN = 1000
    grid = (N // BN,)
    return pl.pallas_call(
        _tc_tail_body,
        grid=grid,
        in_specs=[
            pl.BlockSpec((1, BN, H), lambda i: (0, i, 0)),
            pl.BlockSpec((1, BN, H), lambda i: (1, i, 0)),
            pl.BlockSpec((1, BN, 8), lambda i: (0, i, 0)),
            pl.BlockSpec((1, BN, 8), lambda i: (1, i, 0)),
            pl.BlockSpec((BN, D), lambda i: (i, 0)),
            pl.BlockSpec((D, D), lambda i: (0, 0)),
            pl.BlockSpec((1, D), lambda i: (0, 0)),
            pl.BlockSpec((D, D), lambda i: (0, 0)),
            pl.BlockSpec((1, D), lambda i: (0, 0)),
            pl.BlockSpec((1, D), lambda i: (0, 0)),
        ],
        out_specs=pl.BlockSpec((BN, D), lambda i: (i, 0)),
        out_shape=jax.ShapeDtypeStruct((N, D), jnp.float32),
    )(agg2, agg2, deg2, deg2, x, W_l, b_l.reshape(1, D), W_r,
      gamma.reshape(1, D), beta.reshape(1, D))


def kernel(x, edge_index, W_l, b_l, W_r, ln_gamma, ln_beta):
    N, D = x.shape
    E = edge_index.shape[1]
    H = D // 2
    NS = 16

    sc_agg, NCH, NCH2, ZR, NPAD = _make_sc_agg(N, E, D)
    e_pt = E // NS
    e_dt = E // (2 * NS)
    EP = NCH * _K
    EP2 = NCH2 * _K

    x2 = jnp.concatenate([x[:, :H], x[:, H:]], axis=0)
    pat = jnp.repeat(jnp.eye(_PN, dtype=jnp.float32), 128 // _PN, axis=1)

    src = edge_index[0].reshape(NS, e_pt)
    dst = edge_index[1].reshape(NS, e_pt)
    src_p = jnp.pad(src, ((0, 0), (0, EP - e_pt)))
    dst_p = jnp.pad(dst, ((0, 0), (0, EP - e_pt)), constant_values=N)
    srcs = jnp.stack([src_p, src_p + N]).reshape(2, NS, NCH, _K)
    dsts = dst_p.reshape(NS, NCH, _K)

    dd = edge_index[1].reshape(2, NS, e_dt)
    dd_p = jnp.pad(dd, ((0, 0), (0, 0), (0, EP2 - e_dt)), constant_values=N)
    patd = (dd_p % _PN).reshape(2, NS, NCH2, _K)
    rowd = (dd_p // _PN).reshape(2, NS, NCH2, _K)

    zrow = jnp.zeros((_K, H), jnp.float32)

    agg2, deg8 = sc_agg(x2, pat, srcs, dsts, patd, rowd, zrow)
    deg2 = deg8.reshape(2, NPAD, 128 // _PN)
    return _tc_tail(agg2, deg2, x, W_l, b_l, W_r, ln_gamma, ln_beta)
